# baseline (device time: 2204295 ns/iter reference)
import jax
import jax.numpy as jnp
from jax import lax
from jax.experimental import pallas as pl
from jax.experimental.pallas import tpu as pltpu

R = 1024
K_X = (0, 1, 2)
K_Y = (3, 4, 5)
K_Z = (6, 7)


def kernel(x):
    m, n = x.shape
    qm = m // 4
    ck = qm // R
    nc = m // R

    def body(
        x_ref, out_ref, rbuf, vin, vout,
        in_sems, out_sems, local_sems, send_sems, recv_sems,
    ):
        my_x = lax.axis_index("x")
        my_y = lax.axis_index("y")
        my_z = lax.axis_index("z")

        mine = my_x * m
        other = (1 - my_x) * m
        q_mine = 2 * my_y + my_z
        q_bz = 2 * my_y + (1 - my_z)
        q_by = 2 * (1 - my_y) + my_z
        q_diag = 2 * (1 - my_y) + (1 - my_z)

        A, A2, BZ, BY, CY, CZ = range(6)

        x_nbr = (1 - my_x, my_y, my_z)
        y_nbr = (my_x, 1 - my_y, my_z)
        z_nbr = (my_x, my_y, 1 - my_z)

        def remote(phase, k, src_ref, dst_row, dev):
            return pltpu.make_async_remote_copy(
                src_ref=src_ref,
                dst_ref=rbuf.at[pl.ds(dst_row, R)],
                send_sem=send_sems.at[phase, k],
                recv_sem=recv_sems.at[phase, k],
                device_id=dev,
                device_id_type=pl.DeviceIdType.MESH,
            )

        def land(phase, k, row):
            cp = pltpu.make_async_copy(
                rbuf.at[pl.ds(row, R)],
                out_ref.at[pl.ds(other + row, R)],
                local_sems.at[phase, k],
            )
            cp.start()
            return cp

        rdmas = {}
        lands = []

        cast_quarters = [q_mine, q_diag, q_bz, q_by]

        def cast_row(j):
            return cast_quarters[j // ck] * qm + (j % ck) * R

        def in_copy(j, slot):
            return pltpu.make_async_copy(
                x_ref.at[pl.ds(cast_row(j), R)], vin.at[slot], in_sems.at[slot]
            )

        pending_store = [None, None]
        in_copy(0, 0).start()
        for j in range(nc):
            slot = j % 2
            if j + 1 < nc:
                in_copy(j + 1, 1 - slot).start()
            in_copy(j, slot).wait()
            if pending_store[slot] is not None:
                pending_store[slot].wait()
                pending_store[slot] = None
            vout[slot] = vin[slot].astype(jnp.bfloat16)
            st = pltpu.make_async_copy(
                vout.at[slot],
                out_ref.at[pl.ds(mine + cast_row(j), R)],
                out_sems.at[slot],
            )
            st.start()
            if j < ck:
                st.wait()
                row = q_mine * qm + j * R
                rd = remote(A, j, out_ref.at[pl.ds(mine + row, R)], row, x_nbr)
                rd.start()
                rdmas[(A, j)] = rd
            elif j - ck in K_X:
                st.wait()
                k = j - ck
                row = q_diag * qm + k * R
                rd = remote(A2, k, out_ref.at[pl.ds(mine + row, R)], row, x_nbr)
                rd.start()
                rdmas[(A2, k)] = rd
            else:
                pending_store[slot] = st
        for st in pending_store:
            if st is not None:
                st.wait()

        for k in range(ck):
            rdmas[(A, k)].wait_recv()
            row = q_mine * qm + k * R
            lands.append(land(A, k, row))
            src = rbuf.at[pl.ds(row, R)]
            rd = remote(BZ, k, src, row, z_nbr)
            rd.start()
            rdmas[(BZ, k)] = rd
            rd = remote(BY, k, src, row, y_nbr)
            rd.start()
            rdmas[(BY, k)] = rd

        for k in range(ck):
            rdmas[(BZ, k)].wait_recv()
            row = q_bz * qm + k * R
            lands.append(land(BZ, k, row))
            if k in K_Y:
                rd = remote(CY, k, rbuf.at[pl.ds(row, R)], row, y_nbr)
                rd.start()
                rdmas[(CY, k)] = rd
        for k in range(ck):
            rdmas[(BY, k)].wait_recv()
            row = q_by * qm + k * R
            lands.append(land(BY, k, row))
            if k in K_Z:
                rd = remote(CZ, k, rbuf.at[pl.ds(row, R)], row, z_nbr)
                rd.start()
                rdmas[(CZ, k)] = rd

        for k in K_X:
            rdmas[(A2, k)].wait_recv()
            lands.append(land(A2, k, q_diag * qm + k * R))
        for k in K_Y:
            rdmas[(CY, k)].wait_recv()
            lands.append(land(CY, k, q_diag * qm + k * R))
        for k in K_Z:
            rdmas[(CZ, k)].wait_recv()
            lands.append(land(CZ, k, q_diag * qm + k * R))
        for cp in lands:
            cp.wait()
        for rd in rdmas.values():
            rd.wait_send()

    out, _ = pl.pallas_call(
        body,
        out_shape=(
            jax.ShapeDtypeStruct((2 * m, n), jnp.bfloat16),
            jax.ShapeDtypeStruct((m, n), jnp.bfloat16),
        ),
        in_specs=[pl.BlockSpec(memory_space=pl.ANY)],
        out_specs=(
            pl.BlockSpec(memory_space=pl.ANY),
            pl.BlockSpec(memory_space=pl.ANY),
        ),
        scratch_shapes=[
            pltpu.VMEM((2, R, n), x.dtype),
            pltpu.VMEM((2, R, n), jnp.bfloat16),
            pltpu.SemaphoreType.DMA((2,)),
            pltpu.SemaphoreType.DMA((2,)),
            pltpu.SemaphoreType.DMA((6, 8)),
            pltpu.SemaphoreType.DMA((6, 8)),
            pltpu.SemaphoreType.DMA((6, 8)),
        ],
    )(x)
    return out


# device time: 409374 ns/iter; 5.3846x vs baseline; 5.3846x over previous
import itertools

import jax
import jax.numpy as jnp
from jax import lax
from jax.experimental import pallas as pl
from jax.experimental.pallas import tpu as pltpu

CAST_R = 1024
SIZES = (512,) + (1024,) * 7 + (512,)
OFFS = tuple(itertools.accumulate((0,) + SIZES[:-1]))
ENDS = tuple(o + s for o, s in zip(OFFS, SIZES))
DIAG = ((0, 2736), (2736, 2728), (5464, 2728))
CY_DEP = min(k for k in range(len(SIZES)) if ENDS[k] >= DIAG[1][0] + DIAG[1][1])
CZ_DEP = min(k for k in range(len(SIZES)) if ENDS[k] >= DIAG[2][0] + DIAG[2][1])
A_TRIG = tuple((ENDS[k] - 1) // CAST_R for k in range(len(SIZES)))


def kernel(x):
    m, n = x.shape
    qm = m // 4
    cpq = qm // CAST_R
    nc = m // CAST_R
    nk = len(SIZES)

    def body(x_ref, out_ref, vin, vout, in_sems, out_sems, send_sems, recv_sems):
        my_x = lax.axis_index("x")
        my_y = lax.axis_index("y")
        my_z = lax.axis_index("z")

        mine = my_x * m
        other = (1 - my_x) * m
        q_mine = 2 * my_y + my_z
        q_bz = 2 * my_y + (1 - my_z)
        q_by = 2 * (1 - my_y) + my_z
        q_diag = 2 * (1 - my_y) + (1 - my_z)

        A, A2, BZ, BY, CY, CZ = range(6)

        x_nbr = (1 - my_x, my_y, my_z)
        y_nbr = (my_x, 1 - my_y, my_z)
        z_nbr = (my_x, my_y, 1 - my_z)

        def remote(phase, k, row, sz, dev):
            return pltpu.make_async_remote_copy(
                src_ref=out_ref.at[pl.ds(row, sz)],
                dst_ref=out_ref.at[pl.ds(row, sz)],
                send_sem=send_sems.at[phase, k],
                recv_sem=recv_sems.at[phase, k],
                device_id=dev,
                device_id_type=pl.DeviceIdType.MESH,
            )

        rdmas = {}

        cast_quarters = [q_mine, q_diag, q_bz, q_by]

        def cast_row(j):
            return cast_quarters[j // cpq] * qm + (j % cpq) * CAST_R

        def in_copy(j, slot):
            return pltpu.make_async_copy(
                x_ref.at[pl.ds(cast_row(j), CAST_R)], vin.at[slot], in_sems.at[slot]
            )

        a2_trig = cpq + (DIAG[0][1] - 1) // CAST_R

        pending_store = [None, None]
        in_copy(0, 0).start()
        for j in range(nc):
            slot = j % 2
            if j + 1 < nc:
                in_copy(j + 1, 1 - slot).start()
            in_copy(j, slot).wait()
            if pending_store[slot] is not None:
                pending_store[slot].wait()
                pending_store[slot] = None
            vout[slot] = vin[slot].astype(jnp.bfloat16)
            st = pltpu.make_async_copy(
                vout.at[slot],
                out_ref.at[pl.ds(mine + cast_row(j), CAST_R)],
                out_sems.at[slot],
            )
            st.start()
            a_ready = [k for k in range(nk) if A_TRIG[k] == j] if j < cpq else []
            if a_ready:
                st.wait()
                for k in a_ready:
                    rd = remote(A, k, mine + q_mine * qm + OFFS[k], SIZES[k], x_nbr)
                    rd.start()
                    rdmas[(A, k)] = rd
            elif j == a2_trig:
                st.wait()
                rd = remote(A2, 0, mine + q_diag * qm + DIAG[0][0], DIAG[0][1], x_nbr)
                rd.start()
                rdmas[(A2, 0)] = rd
            else:
                pending_store[slot] = st
        for st in pending_store:
            if st is not None:
                st.wait()

        for k in range(nk):
            rdmas[(A, k)].wait_recv()
            row = other + q_mine * qm + OFFS[k]
            rd = remote(BZ, k, row, SIZES[k], z_nbr)
            rd.start()
            rdmas[(BZ, k)] = rd
            rd = remote(BY, k, row, SIZES[k], y_nbr)
            rd.start()
            rdmas[(BY, k)] = rd

        for k in range(nk):
            rdmas[(BZ, k)].wait_recv()
            if k == CY_DEP:
                rd = remote(CY, 0, other + q_bz * qm + DIAG[1][0], DIAG[1][1], y_nbr)
                rd.start()
                rdmas[(CY, 0)] = rd
        for k in range(nk):
            rdmas[(BY, k)].wait_recv()
            if k == CZ_DEP:
                rd = remote(CZ, 0, other + q_by * qm + DIAG[2][0], DIAG[2][1], z_nbr)
                rd.start()
                rdmas[(CZ, 0)] = rd

        for phase in (A2, CY, CZ):
            rdmas[(phase, 0)].wait_recv()
        for rd in rdmas.values():
            rd.wait_send()

    return pl.pallas_call(
        body,
        out_shape=jax.ShapeDtypeStruct((2 * m, n), jnp.bfloat16),
        in_specs=[pl.BlockSpec(memory_space=pl.ANY)],
        out_specs=pl.BlockSpec(memory_space=pl.ANY),
        scratch_shapes=[
            pltpu.VMEM((2, CAST_R, n), x.dtype),
            pltpu.VMEM((2, CAST_R, n), jnp.bfloat16),
            pltpu.SemaphoreType.DMA((2,)),
            pltpu.SemaphoreType.DMA((2,)),
            pltpu.SemaphoreType.DMA((6, 9)),
            pltpu.SemaphoreType.DMA((6, 9)),
        ],
    )(x)
